# baseline (speedup 1.0000x reference)
import functools

import jax
import jax.numpy as jnp
from jax import lax
from jax.experimental import pallas as pl
from jax.experimental.pallas import tpu as pltpu
from jax.experimental.pallas import tpu_sc as plsc


B, At, Nbr = 2, 256, 16
N_NODE, N_EDGE = 64, 64
ROWS = B * At * Nbr
PAIRS = ROWS // 2
ATOMS = B * At
_NC, _NS = 2, 16
_NW = _NC * _NS
_PER_W = ROWS // _NW
_CH = 128
_NEG = -1e30

_T1G = 16
_AB = ATOMS // _T1G
_PB = _AB * Nbr // 2
_T2R = 256
_T2G = ROWS // _T2R
_T2A = _T2R // Nbr
_GRID = _T1G + _T2G + 1


def _dot(a, b):
    return jax.lax.dot_general(
        a, b, (((1,), (0,)), ((), ())),
        precision=jax.lax.Precision.DEFAULT,
        preferred_element_type=jnp.float32)


def _sc_gather_body(table_hbm, idx2_hbm, out_hbm,
                    idx_v, rows_v, sem_a, sem_b):
    wid = lax.axis_index("s") * _NC + lax.axis_index("c")
    base = wid * _PER_W
    pltpu.sync_copy(idx2_hbm.at[pl.ds(2 * wid, 2)], idx_v)
    ca = pltpu.async_copy(table_hbm.at[idx_v.at[0]],
                          rows_v.at[pl.ds(0, _CH)], sem_a)
    cb = pltpu.async_copy(table_hbm.at[idx_v.at[1]],
                          rows_v.at[pl.ds(_CH, _CH)], sem_b)
    ca.wait()
    cb.wait()
    pltpu.sync_copy(rows_v, out_hbm.at[pl.ds(base, _PER_W)])


@functools.cache
def _sc_gather():
    return pl.kernel(
        _sc_gather_body,
        out_type=jax.ShapeDtypeStruct((ROWS, N_NODE), jnp.float32),
        mesh=plsc.VectorSubcoreMesh(core_axis_name="c", subcore_axis_name="s",
                                    num_cores=_NC, num_subcores=_NS),
        scratch_types=[
            pltpu.VMEM((2, _CH), jnp.int32),
            pltpu.VMEM((_PER_W, N_NODE), jnp.float32),
            pltpu.SemaphoreType.DMA,
            pltpu.SemaphoreType.DMA,
        ],
        compiler_params=pltpu.CompilerParams(use_tc_tiling_on_sc=False),
    )


def _tc_body(node_ref, njp_ref, edgep_ref, mask2_ref,
             idx_ref, nj_ref, edge_ref, mask_ref,
             wc2_ref, wa_ref, wt_n_ref, wt_e_ref, bc2_ref,
             wu_nj_ref, wu_e_ref, bu_ref, gamma2_ref, beta2_ref,
             out_ref,
             tge_s, a_s, basep_s, three_s):
    p = pl.program_id(0)

    @pl.when(p < _T1G)
    def _phase1():
        node = node_ref[...]
        njp = njp_ref[...]
        edgep = edgep_ref[...]
        mask2 = mask2_ref[...]

        lane = lax.broadcasted_iota(jnp.int32, (_PB, 128), 1)
        m_lo = mask2[:, 0:1]
        m_hi = mask2[:, 1:2]
        mfull = jnp.where(lane < 64, m_lo, m_hi)

        njmp = njp * mfull

        ndup = jnp.concatenate([node, node], axis=1)
        prodp = (njmp.reshape(_AB, 8, 128) * ndup[:, None, :]).reshape(_PB, 128)
        c2 = _dot(prodp, wc2_ref[...]) + bc2_ref[...]
        basep_s[pl.ds(p * _PB, _PB)] = (
            edgep + jax.nn.sigmoid(c2[:, :128]) * jnp.tanh(c2[:, 128:]))

        a_s[pl.ds(p * _AB, _AB)] = _dot(node, wa_ref[...])

        tge = _dot(njp, wt_n_ref[...]) + _dot(edgep, wt_e_ref[...])
        lane2 = lax.broadcasted_iota(jnp.int32, (_PB, 256), 1)
        mfull2 = jnp.where(lane2 < 64, m_lo, jnp.where(lane2 < 128, m_hi, 1.0))
        tge_s[pl.ds(p * _AB, _AB)] = (
            (tge + (mfull2 - 1.0) * (-_NEG)).astype(jnp.bfloat16)
            .reshape(_AB, 8, 256))

    @pl.when(jnp.logical_and(p >= _T1G, p < _T1G + _T2G))
    def _phase2():
        q = p - _T1G
        b = q // (_T2G // B)
        idx = idx_ref[...]
        cols = lax.broadcasted_iota(jnp.int32, (_T2R, At), 1)
        oh = jnp.where(idx == cols, 1.0, 0.0).astype(jnp.bfloat16)

        njm = nj_ref[...] * mask_ref[...]
        u = (_dot(njm, wu_nj_ref[...]) + _dot(edge_ref[...], wu_e_ref[...])
             + bu_ref[...])
        ablk = a_s[pl.ds(q * _T2A, _T2A)]
        a3 = jnp.broadcast_to(ablk[:, None, :], (_T2A, Nbr, 256))
        u = u + a3.reshape(_T2R, 256)

        acc = jnp.zeros((_T2R, 128), jnp.float32)
        for kk in range(8):
            slab = tge_s[pl.ds(b * At, At), kk]
            c = _dot(oh, slab) + u
            acc = acc + jax.nn.sigmoid(c[:, :128]) * jnp.tanh(c[:, 128:])
        three_s[pl.ds(q * _T2R, _T2R)] = acc[:, :N_EDGE] + acc[:, N_EDGE:]

    @pl.when(p == _T1G + _T2G)
    def _phase3():
        th = three_s[...].reshape(PAIRS, 2, N_EDGE)
        thp = jnp.concatenate([th[:, 0, :], th[:, 1, :]], axis=1)
        mp = jnp.mean(thp, axis=0, keepdims=True)
        mean = 0.5 * (mp[:, :N_EDGE] + mp[:, N_EDGE:])
        meanf = jnp.concatenate([mean, mean], axis=1)
        cent = thp - meanf
        vp = jnp.mean(cent * cent, axis=0, keepdims=True)
        var = 0.5 * (vp[:, :N_EDGE] + vp[:, N_EDGE:])
        varf = jnp.concatenate([var, var], axis=1)
        normed = (cent * jax.lax.rsqrt(varf + 1e-5) * gamma2_ref[...]
                  + beta2_ref[...])
        out_ref[...] = jnp.tanh(basep_s[...] + normed)


def _tc_call(node, njp, edgep, mask2, idxcol, nj, edge, mask,
             wc2, wa, wt_n, wt_e, bc2, wu_nj, wu_e, bu, gamma2, beta2):
    full = lambda shape: pl.BlockSpec(shape, lambda p: tuple(0 for _ in shape))
    t1m = lambda p: (jnp.minimum(p, _T1G - 1), 0)
    t2m = lambda p: (jnp.clip(p - _T1G, 0, _T2G - 1), 0)
    return pl.pallas_call(
        _tc_body,
        grid=(_GRID,),
        in_specs=[
            pl.BlockSpec((_AB, N_NODE), t1m),
            pl.BlockSpec((_PB, 128), t1m),
            pl.BlockSpec((_PB, 128), t1m),
            pl.BlockSpec((_PB, 2), t1m),
            pl.BlockSpec((_T2R, 1), t2m),
            pl.BlockSpec((_T2R, N_NODE), t2m),
            pl.BlockSpec((_T2R, N_EDGE), t2m),
            pl.BlockSpec((_T2R, 1), t2m),
            full((128, 256)), full((64, 256)), full((128, 256)),
            full((128, 256)), full((1, 256)), full((64, 256)),
            full((64, 256)), full((1, 256)), full((1, 128)), full((1, 128)),
        ],
        out_specs=pl.BlockSpec((PAIRS, 128), lambda p: (0, 0)),
        out_shape=jax.ShapeDtypeStruct((PAIRS, 128), jnp.float32),
        scratch_shapes=[
            pltpu.VMEM((ATOMS, 8, 256), jnp.bfloat16),
            pltpu.VMEM((ATOMS, 256), jnp.float32),
            pltpu.VMEM((PAIRS, 128), jnp.float32),
            pltpu.VMEM((ROWS, N_EDGE), jnp.float32),
        ],
    )(node, njp, edgep, mask2, idxcol, nj, edge, mask,
      wc2, wa, wt_n, wt_e, bc2, wu_nj, wu_e, bu, gamma2, beta2)


def _bd(w):
    z = jnp.zeros((128, 128), dtype=w.dtype)
    return z.at[:64, :64].set(w).at[64:, 64:].set(w)


def kernel(node_embedding, edge_embedding, nbr_idx, nbr_mask,
           W2, b2, W3, b3, bn_gamma, bn_beta):
    assert node_embedding.shape == (B, At, N_NODE)
    assert edge_embedding.shape == (B, At, Nbr, N_EDGE)

    node_flat = node_embedding.reshape(ATOMS, N_NODE)
    edge_flat = edge_embedding.reshape(ROWS, N_EDGE)
    edgep = edge_embedding.reshape(PAIRS, 2 * N_EDGE)
    mask_flat = nbr_mask.reshape(ROWS, 1)
    mask2 = nbr_mask.reshape(PAIRS, 2)
    offs = (jnp.arange(B, dtype=jnp.int32) * At)[:, None, None]
    g_idx = (nbr_idx + offs).reshape(ROWS)

    w2t, w3t = W2.T, W3.T
    w3ni, w3nj, w3nk = w3t[0:64], w3t[64:128], w3t[128:192]
    w3eij, w3ejk = w3t[192:256], w3t[256:320]

    def dup(w):
        return jnp.concatenate([w[:, :64], w[:, :64], w[:, 64:], w[:, 64:]], axis=1)

    wc2 = jnp.concatenate([_bd(w2t[:, :64]), _bd(w2t[:, 64:])], axis=1)
    wt_n = jnp.concatenate([_bd(w3nk[:, :64]), _bd(w3nk[:, 64:])], axis=1)
    wt_e = jnp.concatenate([_bd(w3ejk[:, :64]), _bd(w3ejk[:, 64:])], axis=1)
    wu_nj = dup(w3nj)
    wu_e = dup(w3eij)
    wa = dup(w3ni)
    bc2 = jnp.concatenate([b2[:64], b2[:64], b2[64:], b2[64:]]).reshape(1, 256)
    bu = jnp.concatenate([b3[:64], b3[:64], b3[64:], b3[64:]]).reshape(1, 256)
    gamma2 = jnp.concatenate([bn_gamma, bn_gamma]).reshape(1, 128)
    beta2 = jnp.concatenate([bn_beta, bn_beta]).reshape(1, 128)

    nj = _sc_gather()(node_flat, g_idx.reshape(_NW * 2, _CH))
    njp = nj.reshape(PAIRS, 2 * N_NODE)

    outp = _tc_call(node_flat, njp, edgep, mask2,
                    nbr_idx.reshape(ROWS, 1), nj, edge_flat, mask_flat,
                    wc2, wa, wt_n, wt_e, bc2, wu_nj, wu_e, bu, gamma2, beta2)
    return outp.reshape(B, At, Nbr, N_EDGE)


# speedup vs baseline: 1.4526x; 1.4526x over previous
import functools

import jax
import jax.numpy as jnp
from jax import lax
from jax.experimental import pallas as pl
from jax.experimental.pallas import tpu as pltpu
from jax.experimental.pallas import tpu_sc as plsc


B, At, Nbr = 2, 256, 16
N_NODE, N_EDGE = 64, 64
ROWS = B * At * Nbr
PAIRS = ROWS // 2
ATOMS = B * At
_NC, _NS = 2, 16
_NW = _NC * _NS
_PER_W = ROWS // _NW
_CH = 128
_NEG = -1e30

_T1G = 16
_AB = ATOMS // _T1G
_PB = _AB * Nbr // 2
_T2R = 256
_T2G = ROWS // _T2R
_T2A = _T2R // Nbr
_GRID = _T1G + _T2G + 1


def _dot(a, b):
    return jax.lax.dot_general(
        a, b, (((1,), (0,)), ((), ())),
        precision=jax.lax.Precision.DEFAULT,
        preferred_element_type=jnp.float32)


def _sc_gather_body(table_hbm, idx2_hbm, out_hbm,
                    idx_v, rows_v, sem_a, sem_b):
    wid = lax.axis_index("s") * _NC + lax.axis_index("c")
    base = wid * _PER_W
    pltpu.sync_copy(idx2_hbm.at[pl.ds(2 * wid, 2)], idx_v)
    ca = pltpu.async_copy(table_hbm.at[idx_v.at[0]],
                          rows_v.at[pl.ds(0, _CH)], sem_a)
    cb = pltpu.async_copy(table_hbm.at[idx_v.at[1]],
                          rows_v.at[pl.ds(_CH, _CH)], sem_b)
    ca.wait()
    cb.wait()
    pltpu.sync_copy(rows_v, out_hbm.at[pl.ds(base, _PER_W)])


@functools.cache
def _sc_gather():
    return pl.kernel(
        _sc_gather_body,
        out_type=jax.ShapeDtypeStruct((ROWS, N_NODE), jnp.float32),
        mesh=plsc.VectorSubcoreMesh(core_axis_name="c", subcore_axis_name="s",
                                    num_cores=_NC, num_subcores=_NS),
        scratch_types=[
            pltpu.VMEM((2, _CH), jnp.int32),
            pltpu.VMEM((_PER_W, N_NODE), jnp.float32),
            pltpu.SemaphoreType.DMA,
            pltpu.SemaphoreType.DMA,
        ],
        compiler_params=pltpu.CompilerParams(use_tc_tiling_on_sc=False),
    )


def _tc_body(node_ref, njp_ref, edgep_ref, mask2_ref,
             idx_ref, nj_ref, edge_ref, mask_ref,
             wc2_ref, wa_ref, wt_n_ref, wt_e_ref, bc2_ref,
             wu_nj_ref, wu_e_ref, bu_ref, gamma2_ref, beta2_ref,
             out_ref,
             tge_s, a_s, basep_s, three_s):
    p = pl.program_id(0)

    @pl.when(p < _T1G)
    def _phase1():
        node = node_ref[...]
        njp = njp_ref[...]
        edgep = edgep_ref[...]
        mask2 = mask2_ref[...]

        lane = lax.broadcasted_iota(jnp.int32, (_PB, 128), 1)
        m_lo = mask2[:, 0:1]
        m_hi = mask2[:, 1:2]
        mfull = jnp.where(lane < 64, m_lo, m_hi)

        njmp = njp * mfull

        ndup = jnp.concatenate([node, node], axis=1)
        prodp = (njmp.reshape(_AB, 8, 128) * ndup[:, None, :]).reshape(_PB, 128)
        c2 = _dot(prodp, wc2_ref[...]) + bc2_ref[...]
        basep_s[pl.ds(p * _PB, _PB)] = (
            edgep + jax.nn.sigmoid(c2[:, :128]) * jnp.tanh(c2[:, 128:]))

        a_s[pl.ds(p * _AB, _AB)] = _dot(node, wa_ref[...])

        tge = _dot(njp, wt_n_ref[...]) + _dot(edgep, wt_e_ref[...])
        lane2 = lax.broadcasted_iota(jnp.int32, (_PB, 256), 1)
        mfull2 = jnp.where(lane2 < 64, m_lo, jnp.where(lane2 < 128, m_hi, 1.0))
        tge3 = ((tge + (mfull2 - 1.0) * (-_NEG)).astype(jnp.bfloat16)
                .reshape(_AB, 8, 256))
        for kk in range(8):
            tge_s[pl.ds(p * _AB, _AB), pl.ds(kk * 256, 256)] = tge3[:, kk, :]

    @pl.when(jnp.logical_and(p >= _T1G, p < _T1G + _T2G))
    def _phase2():
        q = p - _T1G
        b = q // (_T2G // B)
        idx = idx_ref[...]
        cols = lax.broadcasted_iota(jnp.int32, (_T2R, At), 1)
        oh = jnp.where(idx == cols, 1.0, 0.0).astype(jnp.bfloat16)

        njm = nj_ref[...] * mask_ref[...]
        u = (_dot(njm, wu_nj_ref[...]) + _dot(edge_ref[...], wu_e_ref[...])
             + bu_ref[...])
        ablk = a_s[pl.ds(q * _T2A, _T2A)]
        a3 = jnp.broadcast_to(ablk[:, None, :], (_T2A, Nbr, 256))
        u = u + a3.reshape(_T2R, 256)

        acc = jnp.zeros((_T2R, 128), jnp.float32)
        for kk in range(8):
            slab = tge_s[pl.ds(b * At, At), pl.ds(kk * 256, 256)]
            c = _dot(oh, slab) + u
            acc = acc + jax.nn.sigmoid(c[:, :128]) * jnp.tanh(c[:, 128:])
        three_s[pl.ds(q * _T2R, _T2R)] = acc[:, :N_EDGE] + acc[:, N_EDGE:]

    @pl.when(p == _T1G + _T2G)
    def _phase3():
        th = three_s[...].reshape(PAIRS, 2, N_EDGE)
        thp = jnp.concatenate([th[:, 0, :], th[:, 1, :]], axis=1)
        mp = jnp.mean(thp, axis=0, keepdims=True)
        mean = 0.5 * (mp[:, :N_EDGE] + mp[:, N_EDGE:])
        meanf = jnp.concatenate([mean, mean], axis=1)
        cent = thp - meanf
        vp = jnp.mean(cent * cent, axis=0, keepdims=True)
        var = 0.5 * (vp[:, :N_EDGE] + vp[:, N_EDGE:])
        varf = jnp.concatenate([var, var], axis=1)
        normed = (cent * jax.lax.rsqrt(varf + 1e-5) * gamma2_ref[...]
                  + beta2_ref[...])
        out_ref[...] = jnp.tanh(basep_s[...] + normed)


def _tc_call(node, njp, edgep, mask2, idxcol, nj, edge, mask,
             wc2, wa, wt_n, wt_e, bc2, wu_nj, wu_e, bu, gamma2, beta2):
    full = lambda shape: pl.BlockSpec(shape, lambda p: tuple(0 for _ in shape))
    t1m = lambda p: (jnp.minimum(p, _T1G - 1), 0)
    t2m = lambda p: (jnp.clip(p - _T1G, 0, _T2G - 1), 0)
    return pl.pallas_call(
        _tc_body,
        grid=(_GRID,),
        in_specs=[
            pl.BlockSpec((_AB, N_NODE), t1m),
            pl.BlockSpec((_PB, 128), t1m),
            pl.BlockSpec((_PB, 128), t1m),
            pl.BlockSpec((_PB, 2), t1m),
            pl.BlockSpec((_T2R, 1), t2m),
            pl.BlockSpec((_T2R, N_NODE), t2m),
            pl.BlockSpec((_T2R, N_EDGE), t2m),
            pl.BlockSpec((_T2R, 1), t2m),
            full((128, 256)), full((64, 256)), full((128, 256)),
            full((128, 256)), full((1, 256)), full((64, 256)),
            full((64, 256)), full((1, 256)), full((1, 128)), full((1, 128)),
        ],
        out_specs=pl.BlockSpec((PAIRS, 128), lambda p: (0, 0)),
        out_shape=jax.ShapeDtypeStruct((PAIRS, 128), jnp.float32),
        scratch_shapes=[
            pltpu.VMEM((ATOMS, 8 * 256), jnp.bfloat16),
            pltpu.VMEM((ATOMS, 256), jnp.float32),
            pltpu.VMEM((PAIRS, 128), jnp.float32),
            pltpu.VMEM((ROWS, N_EDGE), jnp.float32),
        ],
    )(node, njp, edgep, mask2, idxcol, nj, edge, mask,
      wc2, wa, wt_n, wt_e, bc2, wu_nj, wu_e, bu, gamma2, beta2)


def _bd(w):
    z = jnp.zeros((128, 128), dtype=w.dtype)
    return z.at[:64, :64].set(w).at[64:, 64:].set(w)


def kernel(node_embedding, edge_embedding, nbr_idx, nbr_mask,
           W2, b2, W3, b3, bn_gamma, bn_beta):
    assert node_embedding.shape == (B, At, N_NODE)
    assert edge_embedding.shape == (B, At, Nbr, N_EDGE)

    node_flat = node_embedding.reshape(ATOMS, N_NODE)
    edge_flat = edge_embedding.reshape(ROWS, N_EDGE)
    edgep = edge_embedding.reshape(PAIRS, 2 * N_EDGE)
    mask_flat = nbr_mask.reshape(ROWS, 1)
    mask2 = nbr_mask.reshape(PAIRS, 2)
    offs = (jnp.arange(B, dtype=jnp.int32) * At)[:, None, None]
    g_idx = (nbr_idx + offs).reshape(ROWS)

    w2t, w3t = W2.T, W3.T
    w3ni, w3nj, w3nk = w3t[0:64], w3t[64:128], w3t[128:192]
    w3eij, w3ejk = w3t[192:256], w3t[256:320]

    def dup(w):
        return jnp.concatenate([w[:, :64], w[:, :64], w[:, 64:], w[:, 64:]], axis=1)

    wc2 = jnp.concatenate([_bd(w2t[:, :64]), _bd(w2t[:, 64:])], axis=1)
    wt_n = jnp.concatenate([_bd(w3nk[:, :64]), _bd(w3nk[:, 64:])], axis=1)
    wt_e = jnp.concatenate([_bd(w3ejk[:, :64]), _bd(w3ejk[:, 64:])], axis=1)
    wu_nj = dup(w3nj)
    wu_e = dup(w3eij)
    wa = dup(w3ni)
    bc2 = jnp.concatenate([b2[:64], b2[:64], b2[64:], b2[64:]]).reshape(1, 256)
    bu = jnp.concatenate([b3[:64], b3[:64], b3[64:], b3[64:]]).reshape(1, 256)
    gamma2 = jnp.concatenate([bn_gamma, bn_gamma]).reshape(1, 128)
    beta2 = jnp.concatenate([bn_beta, bn_beta]).reshape(1, 128)

    nj = _sc_gather()(node_flat, g_idx.reshape(_NW * 2, _CH))
    njp = nj.reshape(PAIRS, 2 * N_NODE)

    outp = _tc_call(node_flat, njp, edgep, mask2,
                    nbr_idx.reshape(ROWS, 1), nj, edge_flat, mask_flat,
                    wc2, wa, wt_n, wt_e, bc2, wu_nj, wu_e, bu, gamma2, beta2)
    return outp.reshape(B, At, Nbr, N_EDGE)


# speedup vs baseline: 1.4554x; 1.0020x over previous
import functools

import jax
import jax.numpy as jnp
from jax import lax
from jax.experimental import pallas as pl
from jax.experimental.pallas import tpu as pltpu
from jax.experimental.pallas import tpu_sc as plsc


B, At, Nbr = 2, 256, 16
N_NODE, N_EDGE = 64, 64
ROWS = B * At * Nbr
PAIRS = ROWS // 2
ATOMS = B * At
_NC, _NS = 2, 16
_NW = _NC * _NS
_PER_W = ROWS // _NW
_CH = 128
_NEG = -1e30

_T1G = 16
_AB = ATOMS // _T1G
_PB = _AB * Nbr // 2
_T2R = 256
_T2G = ROWS // _T2R
_T2A = _T2R // Nbr
_GRID = _T1G + _T2G + 1


def _dot(a, b):
    return jax.lax.dot_general(
        a, b, (((1,), (0,)), ((), ())),
        precision=jax.lax.Precision.DEFAULT,
        preferred_element_type=jnp.float32)


def _sc_gather_body(table_hbm, idx2_hbm, out_hbm,
                    idx_v, rows_v, sem_a, sem_b):
    wid = lax.axis_index("s") * _NC + lax.axis_index("c")
    base = wid * _PER_W
    pltpu.sync_copy(idx2_hbm.at[pl.ds(2 * wid, 2)], idx_v)
    ca = pltpu.async_copy(table_hbm.at[idx_v.at[0]],
                          rows_v.at[pl.ds(0, _CH)], sem_a)
    cb = pltpu.async_copy(table_hbm.at[idx_v.at[1]],
                          rows_v.at[pl.ds(_CH, _CH)], sem_b)
    ca.wait()
    cb.wait()
    pltpu.sync_copy(rows_v, out_hbm.at[pl.ds(base, _PER_W)])


@functools.cache
def _sc_gather():
    return pl.kernel(
        _sc_gather_body,
        out_type=jax.ShapeDtypeStruct((ROWS, N_NODE), jnp.float32),
        mesh=plsc.VectorSubcoreMesh(core_axis_name="c", subcore_axis_name="s",
                                    num_cores=_NC, num_subcores=_NS),
        scratch_types=[
            pltpu.VMEM((2, _CH), jnp.int32),
            pltpu.VMEM((_PER_W, N_NODE), jnp.float32),
            pltpu.SemaphoreType.DMA,
            pltpu.SemaphoreType.DMA,
        ],
        compiler_params=pltpu.CompilerParams(use_tc_tiling_on_sc=False),
    )


def _tc_body(node_ref, njp_ref, edgep_ref, mask2_ref,
             idx_ref, nj_ref, edge_ref, mask_ref,
             wc2_ref, wa_ref, wt_n_ref, wt_e_ref, bc2_ref,
             wu_nj_ref, wu_e_ref, bu_ref, gamma2_ref, beta2_ref,
             out_ref,
             tge_s, a_s, basep_s, three_s, oh_s):
    p = pl.program_id(0)

    @pl.when(p < _T1G)
    def _phase1():
        node = node_ref[...]
        njp = njp_ref[...]
        edgep = edgep_ref[...]
        mask2 = mask2_ref[...]

        lane = lax.broadcasted_iota(jnp.int32, (_PB, 128), 1)
        m_lo = mask2[:, 0:1]
        m_hi = mask2[:, 1:2]
        mfull = jnp.where(lane < 64, m_lo, m_hi)

        njmp = njp * mfull

        ndup = jnp.concatenate([node, node], axis=1)
        prodp = (njmp.reshape(_AB, 8, 128) * ndup[:, None, :]).reshape(_PB, 128)
        c2 = _dot(prodp, wc2_ref[...]) + bc2_ref[...]
        basep_s[pl.ds(p * _PB, _PB)] = (
            edgep + jax.nn.sigmoid(c2[:, :128]) * jnp.tanh(c2[:, 128:]))

        a_s[pl.ds(p * _AB, _AB)] = _dot(node, wa_ref[...])

        tge = _dot(njp, wt_n_ref[...]) + _dot(edgep, wt_e_ref[...])
        lane2 = lax.broadcasted_iota(jnp.int32, (_PB, 256), 1)
        mfull2 = jnp.where(lane2 < 64, m_lo, jnp.where(lane2 < 128, m_hi, 1.0))
        tge3 = ((tge + (mfull2 - 1.0) * (-_NEG)).astype(jnp.bfloat16)
                .reshape(_AB, 8, 256))
        for kk in range(8):
            tge_s[pl.ds(p * _AB, _AB), pl.ds(kk * 256, 256)] = tge3[:, kk, :]

        idxb = idx_ref[...]
        colsb = lax.broadcasted_iota(jnp.int32, (2 * _PB, At), 1)
        oh_s[pl.ds(p * 2 * _PB, 2 * _PB)] = (
            jnp.where(idxb == colsb, 1.0, 0.0).astype(jnp.bfloat16))

    @pl.when(jnp.logical_and(p >= _T1G, p < _T1G + _T2G))
    def _phase2():
        q = p - _T1G
        b = q // (_T2G // B)
        oh = oh_s[pl.ds(q * _T2R, _T2R)]

        njm = nj_ref[...] * mask_ref[...]
        u = (_dot(njm, wu_nj_ref[...]) + _dot(edge_ref[...], wu_e_ref[...])
             + bu_ref[...])
        ablk = a_s[pl.ds(q * _T2A, _T2A)]
        a3 = jnp.broadcast_to(ablk[:, None, :], (_T2A, Nbr, 256))
        u = u + a3.reshape(_T2R, 256)

        acc = jnp.zeros((_T2R, 128), jnp.float32)
        for kk in range(8):
            slab = tge_s[pl.ds(b * At, At), pl.ds(kk * 256, 256)]
            c = _dot(oh, slab) + u
            acc = acc + jax.nn.sigmoid(c[:, :128]) * jnp.tanh(c[:, 128:])
        three_s[pl.ds(q * _T2R, _T2R)] = acc[:, :N_EDGE] + acc[:, N_EDGE:]

    @pl.when(p == _T1G + _T2G)
    def _phase3():
        th = three_s[...].reshape(PAIRS, 2, N_EDGE)
        thp = jnp.concatenate([th[:, 0, :], th[:, 1, :]], axis=1)
        mp = jnp.mean(thp, axis=0, keepdims=True)
        mean = 0.5 * (mp[:, :N_EDGE] + mp[:, N_EDGE:])
        meanf = jnp.concatenate([mean, mean], axis=1)
        cent = thp - meanf
        vp = jnp.mean(cent * cent, axis=0, keepdims=True)
        var = 0.5 * (vp[:, :N_EDGE] + vp[:, N_EDGE:])
        varf = jnp.concatenate([var, var], axis=1)
        normed = (cent * jax.lax.rsqrt(varf + 1e-5) * gamma2_ref[...]
                  + beta2_ref[...])
        out_ref[...] = jnp.tanh(basep_s[...] + normed)


def _tc_call(node, njp, edgep, mask2, idxcol, nj, edge, mask,
             wc2, wa, wt_n, wt_e, bc2, wu_nj, wu_e, bu, gamma2, beta2):
    full = lambda shape: pl.BlockSpec(shape, lambda p: tuple(0 for _ in shape))
    t1m = lambda p: (jnp.minimum(p, _T1G - 1), 0)
    t2m = lambda p: (jnp.clip(p - _T1G, 0, _T2G - 1), 0)
    return pl.pallas_call(
        _tc_body,
        grid=(_GRID,),
        in_specs=[
            pl.BlockSpec((_AB, N_NODE), t1m),
            pl.BlockSpec((_PB, 128), t1m),
            pl.BlockSpec((_PB, 128), t1m),
            pl.BlockSpec((_PB, 2), t1m),
            pl.BlockSpec((2 * _PB, 1), t1m),
            pl.BlockSpec((_T2R, N_NODE), t2m),
            pl.BlockSpec((_T2R, N_EDGE), t2m),
            pl.BlockSpec((_T2R, 1), t2m),
            full((128, 256)), full((64, 256)), full((128, 256)),
            full((128, 256)), full((1, 256)), full((64, 256)),
            full((64, 256)), full((1, 256)), full((1, 128)), full((1, 128)),
        ],
        out_specs=pl.BlockSpec((PAIRS, 128), lambda p: (0, 0)),
        out_shape=jax.ShapeDtypeStruct((PAIRS, 128), jnp.float32),
        scratch_shapes=[
            pltpu.VMEM((ATOMS, 8 * 256), jnp.bfloat16),
            pltpu.VMEM((ATOMS, 256), jnp.float32),
            pltpu.VMEM((PAIRS, 128), jnp.float32),
            pltpu.VMEM((ROWS, N_EDGE), jnp.float32),
            pltpu.VMEM((ROWS, At), jnp.bfloat16),
        ],
    )(node, njp, edgep, mask2, idxcol, nj, edge, mask,
      wc2, wa, wt_n, wt_e, bc2, wu_nj, wu_e, bu, gamma2, beta2)


def _bd(w):
    z = jnp.zeros((128, 128), dtype=w.dtype)
    return z.at[:64, :64].set(w).at[64:, 64:].set(w)


def kernel(node_embedding, edge_embedding, nbr_idx, nbr_mask,
           W2, b2, W3, b3, bn_gamma, bn_beta):
    assert node_embedding.shape == (B, At, N_NODE)
    assert edge_embedding.shape == (B, At, Nbr, N_EDGE)

    node_flat = node_embedding.reshape(ATOMS, N_NODE)
    edge_flat = edge_embedding.reshape(ROWS, N_EDGE)
    edgep = edge_embedding.reshape(PAIRS, 2 * N_EDGE)
    mask_flat = nbr_mask.reshape(ROWS, 1)
    mask2 = nbr_mask.reshape(PAIRS, 2)
    offs = (jnp.arange(B, dtype=jnp.int32) * At)[:, None, None]
    g_idx = (nbr_idx + offs).reshape(ROWS)

    w2t, w3t = W2.T, W3.T
    w3ni, w3nj, w3nk = w3t[0:64], w3t[64:128], w3t[128:192]
    w3eij, w3ejk = w3t[192:256], w3t[256:320]

    def dup(w):
        return jnp.concatenate([w[:, :64], w[:, :64], w[:, 64:], w[:, 64:]], axis=1)

    wc2 = jnp.concatenate([_bd(w2t[:, :64]), _bd(w2t[:, 64:])], axis=1)
    wt_n = jnp.concatenate([_bd(w3nk[:, :64]), _bd(w3nk[:, 64:])], axis=1)
    wt_e = jnp.concatenate([_bd(w3ejk[:, :64]), _bd(w3ejk[:, 64:])], axis=1)
    wu_nj = dup(w3nj)
    wu_e = dup(w3eij)
    wa = dup(w3ni)
    bc2 = jnp.concatenate([b2[:64], b2[:64], b2[64:], b2[64:]]).reshape(1, 256)
    bu = jnp.concatenate([b3[:64], b3[:64], b3[64:], b3[64:]]).reshape(1, 256)
    gamma2 = jnp.concatenate([bn_gamma, bn_gamma]).reshape(1, 128)
    beta2 = jnp.concatenate([bn_beta, bn_beta]).reshape(1, 128)

    nj = _sc_gather()(node_flat, g_idx.reshape(_NW * 2, _CH))
    njp = nj.reshape(PAIRS, 2 * N_NODE)

    outp = _tc_call(node_flat, njp, edgep, mask2,
                    nbr_idx.reshape(ROWS, 1), nj, edge_flat, mask_flat,
                    wc2, wa, wt_n, wt_e, bc2, wu_nj, wu_e, bu, gamma2, beta2)
    return outp.reshape(B, At, Nbr, N_EDGE)
